# baseline (speedup 1.0000x reference)
import functools

import jax
import jax.numpy as jnp
from jax import lax
from jax.experimental import pallas as pl
from jax.experimental.pallas import tpu as pltpu
from jax.experimental.pallas import tpu_sc as plsc

N = 10000
E = 320000
D = 128
GSZ = 64
G = E // GSZ
NC = 2
NS = 16
NW = NC * NS
ROWS_PER_TILE = (N // NS) // 8 * 8
VECS = D // 16


K = 8
CH = K * GSZ
BODY = 2 * K
GMAX = 160
GTAIL = G - (NW - 1) * GMAX


def _edge_body(compute_deg, a_hbm, b_hbm, edge_hbm, *refs):
    if compute_deg:
        (h_out, deg_out, ic0, ic1, a0, b0, a1, b1, dw0, dw1,
         deg_v, h_shared, ii0, ii1, sg0, sg1, ss0, ss1) = refs
    else:
        (h_out, ic0, ic1, a0, b0, a1, b1, dw0, dw1,
         h_shared, ii0, ii1, sg0, sg1, ss0, ss1) = refs
        deg_out = deg_v = None

    c = lax.axis_index("c")
    s = lax.axis_index("s")
    wid = s * NC + c

    zeros16 = jnp.zeros((16,), jnp.float32)
    ones16 = jnp.ones((16,), jnp.float32)

    def zrow(r, _):
        for v in range(VECS):
            a0[r, pl.ds(v * 16, 16)] = zeros16
        return 0
    lax.fori_loop(0, GSZ, zrow, 0)
    base = pl.multiple_of(s * ROWS_PER_TILE, 8)
    for k in range(ROWS_PER_TILE // GSZ):
        pltpu.sync_copy(a0, h_shared.at[pl.ds(base + k * GSZ, GSZ)])
    rem = ROWS_PER_TILE % GSZ
    if rem:
        pltpu.sync_copy(a0.at[pl.ds(0, rem)],
                        h_shared.at[pl.ds(base + (ROWS_PER_TILE // GSZ) * GSZ, rem)])
    @pl.when(s == NS - 1)
    def _zero_tail():
        pltpu.sync_copy(a0.at[pl.ds(0, N - NS * ROWS_PER_TILE)],
                        h_shared.at[pl.ds(NS * ROWS_PER_TILE, N - NS * ROWS_PER_TILE)])

    if compute_deg:
        def zdeg(i, _):
            deg_v[pl.ds(pl.multiple_of(i * 16, 16), 16)] = zeros16
            return 0
        lax.fori_loop(0, N // 16, zdeg, 0)

    plsc.subcore_barrier()

    start = wid * GMAX
    nb = jnp.where(wid < NW - 1, GMAX // BODY, GTAIL // BODY)

    slots = ((a0, b0, dw0, sg0, ss0), (a1, b1, dw1, sg1, ss1))
    chunks = ((ic0, ii0), (ic1, ii1))

    def chunk_off(cc):
        return pl.multiple_of((start + cc * K) * GSZ, 8)

    def issue_ic(cc, ch):
        ic, sem = ch
        pltpu.async_copy(edge_hbm.at[:, pl.ds(chunk_off(cc), CH)], ic, sem)

    def wait_ic(cc, ch):
        ic, sem = ch
        pltpu.make_async_copy(edge_hbm.at[:, pl.ds(chunk_off(cc), CH)], ic, sem).wait()

    def issue_g(ch, jj, sl):
        ic, _ = ch
        a_buf, b_buf, _, sg, _ = sl
        pltpu.async_copy(a_hbm.at[ic.at[1, pl.ds(jj * GSZ, GSZ)]], a_buf, sg)
        pltpu.async_copy(b_hbm.at[ic.at[0, pl.ds(jj * GSZ, GSZ)]], b_buf, sg)

    def wait_g(ch, jj, sl):
        ic, _ = ch
        a_buf, b_buf, _, sg, _ = sl
        pltpu.make_async_copy(a_hbm.at[ic.at[1, pl.ds(jj * GSZ, GSZ)]], a_buf, sg).wait()
        pltpu.make_async_copy(b_hbm.at[ic.at[0, pl.ds(jj * GSZ, GSZ)]], b_buf, sg).wait()

    def wait_s(sl):
        a_buf, _, dw, _, ss = sl
        pltpu.make_async_copy(a_buf, h_shared.at[dw], ss).wait()

    def fill_dw(ch, jj, sl):
        ic, _ = ch
        dw = sl[2]
        for v in range(GSZ // 16):
            idx16 = ic[1, pl.ds(jj * GSZ + v * 16, 16)]
            dw[pl.ds(v * 16, 16)] = idx16
            if compute_deg:
                plsc.addupdate_scatter(deg_v, [idx16], ones16)

    def compute_scatter(sl):
        a_buf, b_buf, dw, _, ss = sl
        def row4(r, _):
            for rr in range(4):
                for v in range(VECS):
                    s_ = pl.ds(v * 16, 16)
                    a_buf[4 * r + rr, s_] = jnp.maximum(
                        a_buf[4 * r + rr, s_] + b_buf[4 * r + rr, s_], 0.0)
            return 0
        lax.fori_loop(0, GSZ // 4, row4, 0)
        pltpu.async_copy(a_buf, h_shared.at[dw], ss, add=True)

    issue_ic(0, chunks[0])
    issue_ic(1, chunks[1])
    wait_ic(0, chunks[0])
    issue_g(chunks[0], 0, slots[0])

    def body(q, _):
        for j in range(BODY):
            ch = chunks[0] if j < K else chunks[1]
            jj = j % K
            sl = slots[j % 2]
            wait_g(ch, jj, sl)
            if j == K:
                @pl.when(q + 1 < nb)
                def _pfc0():
                    issue_ic(2 * q + 2, chunks[0])
            fill_dw(ch, jj, sl)
            if j + 1 < BODY:
                nsl = slots[(j + 1) % 2]
                if j == 0:
                    @pl.when(q > 0)
                    def _dr():
                        wait_s(nsl)
                else:
                    wait_s(nsl)
                nch = chunks[0] if j + 1 < K else chunks[1]
                issue_g(nch, (j + 1) % K, nsl)
            else:
                @pl.when(q + 1 < nb)
                def _nextbody():
                    wait_ic(2 * q + 2, chunks[0])
                    wait_s(slots[0])
                    issue_g(chunks[0], 0, slots[0])
            compute_scatter(sl)
        @pl.when(q + 1 < nb)
        def _pfc1():
            issue_ic(2 * q + 3, chunks[1])
        return 0

    lax.fori_loop(0, nb, body, 0)

    @pl.when(wid == NW - 1)
    def _epilogue():
        issue_ic(4, chunks[0])
        wait_ic(4, chunks[0])
        for j in range(K):
            sl = slots[j % 2]
            wait_s(sl)
            issue_g(chunks[0], j, sl)
            wait_g(chunks[0], j, sl)
            fill_dw(chunks[0], j, sl)
            compute_scatter(sl)

    wait_s(slots[0])
    wait_s(slots[1])

    plsc.subcore_barrier()
    pltpu.sync_copy(h_shared.at[pl.ds(base, ROWS_PER_TILE)],
                    h_out.at[c].at[pl.ds(base, ROWS_PER_TILE)])
    @pl.when(s == NS - 1)
    def _write_tail():
        tail = N - NS * ROWS_PER_TILE
        pltpu.sync_copy(h_shared.at[pl.ds(NS * ROWS_PER_TILE, tail)],
                        h_out.at[c].at[pl.ds(NS * ROWS_PER_TILE, tail)])
    if compute_deg:
        pltpu.sync_copy(deg_v, deg_out.at[wid])


def _make_edge_kernel(compute_deg):
    mesh = plsc.VectorSubcoreMesh(core_axis_name="c", subcore_axis_name="s",
                                  num_cores=NC, num_subcores=NS)
    out_type = [jax.ShapeDtypeStruct((NC, N, D), jnp.float32)]
    if compute_deg:
        out_type.append(jax.ShapeDtypeStruct((NW, N), jnp.float32))
    scratch = [
        pltpu.VMEM((2, CH), jnp.int32),
        pltpu.VMEM((2, CH), jnp.int32),
        pltpu.VMEM((GSZ, D), jnp.float32),
        pltpu.VMEM((GSZ, D), jnp.float32),
        pltpu.VMEM((GSZ, D), jnp.float32),
        pltpu.VMEM((GSZ, D), jnp.float32),
        pltpu.VMEM((GSZ,), jnp.int32),
        pltpu.VMEM((GSZ,), jnp.int32),
    ]
    if compute_deg:
        scratch.append(pltpu.VMEM((N,), jnp.float32))
    scratch.append(pltpu.VMEM_SHARED((N, D), jnp.float32))
    scratch += [pltpu.SemaphoreType.DMA] * 6
    return pl.kernel(
        functools.partial(_edge_body, compute_deg),
        out_type=tuple(out_type) if compute_deg else out_type[0],
        mesh=mesh,
        scratch_types=scratch,
        compiler_params=pltpu.CompilerParams(needs_layout_passes=False),
    )


_BM = 1000


def _proj_body(x_ref, w_ref, b_ref, oa_ref, ob_ref):
    p = jnp.dot(x_ref[...], w_ref[...],
                preferred_element_type=jnp.float32,
                precision=lax.Precision.HIGHEST) + b_ref[...]
    oa_ref[...] = p[:, :D]
    ob_ref[...] = p[:, D:]


def _proj(x, w, bias):
    return pl.pallas_call(
        _proj_body,
        grid=(N // _BM,),
        in_specs=[
            pl.BlockSpec((_BM, D), lambda i: (i, 0)),
            pl.BlockSpec((D, 2 * D), lambda i: (0, 0)),
            pl.BlockSpec((1, 2 * D), lambda i: (0, 0)),
        ],
        out_specs=[pl.BlockSpec((_BM, D), lambda i: (i, 0))] * 2,
        out_shape=[jax.ShapeDtypeStruct((N, D), jnp.float32)] * 2,
    )(x, w, bias.reshape(1, 2 * D))


def _degsum_body(deg_ref, o_ref):
    o_ref[...] = jnp.sum(deg_ref[...], axis=0)[:, None]


def _degsum(deg):
    return pl.pallas_call(
        _degsum_body,
        grid=(1,),
        in_specs=[pl.BlockSpec((NW, N), lambda i: (0, 0))],
        out_specs=pl.BlockSpec((N, 1), lambda i: (0, 0)),
        out_shape=jax.ShapeDtypeStruct((N, 1), jnp.float32),
    )(deg)


def _combine_body(split, h_ref, deg_ref, w_ref, u_ref, b_ref, *o_refs):
    hs = h_ref[0] + h_ref[1]
    p = (jnp.dot(hs, w_ref[...],
                 preferred_element_type=jnp.float32,
                 precision=lax.Precision.HIGHEST)
         + deg_ref[...] * u_ref[...]
         + b_ref[...])
    if split:
        o_refs[0][...] = p[:, :D]
        o_refs[1][...] = p[:, D:]
    else:
        o_refs[0][...] = p


def _combine(h_stack, deg, w, u, bias):
    k = w.shape[1]
    split = k == 2 * D
    out_specs = [pl.BlockSpec((_BM, D), lambda i: (i, 0))]
    out_shape = [jax.ShapeDtypeStruct((N, D), jnp.float32)]
    if split:
        out_specs = out_specs * 2
        out_shape = out_shape * 2
    res = pl.pallas_call(
        functools.partial(_combine_body, split),
        grid=(N // _BM,),
        in_specs=[
            pl.BlockSpec((NC, _BM, D), lambda i: (0, i, 0)),
            pl.BlockSpec((_BM, 1), lambda i: (i, 0)),
            pl.BlockSpec((D, k), lambda i: (0, 0)),
            pl.BlockSpec((1, k), lambda i: (0, 0)),
            pl.BlockSpec((1, k), lambda i: (0, 0)),
        ],
        out_specs=out_specs,
        out_shape=out_shape,
    )(h_stack, deg, w, u.reshape(1, k), bias.reshape(1, k))
    return res if split else res[0]


def kernel(x, edge_index, W1_0, b1_0, W2_0, b2_0, W1_1, b1_1, W2_1, b2_1):
    Wa0, Wb0 = W1_0[:D], W1_0[D:]
    Wc0 = jnp.concatenate([Wa0 - Wb0, Wb0], axis=1)
    bias0 = jnp.concatenate([b1_0, jnp.zeros_like(b1_0)])
    A0, B0 = _proj(x, Wc0, bias0)

    H0, deg_parts = _make_edge_kernel(True)(A0, B0, edge_index)
    deg = _degsum(deg_parts)

    Wa1, Wb1 = W1_1[:D], W1_1[D:]
    Wcat1 = jnp.concatenate([Wa1 - Wb1, Wb1], axis=1)
    Wc1 = W2_0 @ Wcat1
    u1 = b2_0 @ Wcat1
    bias1 = jnp.concatenate([b1_1, jnp.zeros_like(b1_1)])
    A1, B1 = _combine(H0, deg, Wc1, u1, bias1)

    H1 = _make_edge_kernel(False)(A1, B1, edge_index)

    return _combine(H1, deg, W2_1, b2_1, jnp.zeros_like(b2_1))


# speedup vs baseline: 1.0170x; 1.0170x over previous
import functools

import jax
import jax.numpy as jnp
from jax import lax
from jax.experimental import pallas as pl
from jax.experimental.pallas import tpu as pltpu
from jax.experimental.pallas import tpu_sc as plsc

N = 10000
E = 320000
D = 128
GSZ = 64
G = E // GSZ
NC = 2
NS = 16
NW = NC * NS
ROWS_PER_TILE = (N // NS) // 8 * 8
VECS = D // 16


K = 8
CH = K * GSZ
BODY = 2 * K
GMAX = 160
GTAIL = G - (NW - 1) * GMAX


def _edge_body(compute_deg, a_hbm, b_hbm, edge_hbm, *refs):
    if compute_deg:
        (h_out, deg_out, ic0, ic1, a0, b0, a1, b1, dw0, dw1,
         deg_v, h_shared, ii0, ii1, sg0, sg1, ss0, ss1) = refs
    else:
        (h_out, ic0, ic1, a0, b0, a1, b1, dw0, dw1,
         h_shared, ii0, ii1, sg0, sg1, ss0, ss1) = refs
        deg_out = deg_v = None

    c = lax.axis_index("c")
    s = lax.axis_index("s")
    wid = s * NC + c

    zeros16 = jnp.zeros((16,), jnp.float32)
    ones16 = jnp.ones((16,), jnp.float32)

    def zrow(r, _):
        for v in range(VECS):
            a0[r, pl.ds(v * 16, 16)] = zeros16
        return 0
    lax.fori_loop(0, GSZ, zrow, 0)
    base = pl.multiple_of(s * ROWS_PER_TILE, 8)
    for k in range(ROWS_PER_TILE // GSZ):
        pltpu.sync_copy(a0, h_shared.at[pl.ds(base + k * GSZ, GSZ)])
    rem = ROWS_PER_TILE % GSZ
    if rem:
        pltpu.sync_copy(a0.at[pl.ds(0, rem)],
                        h_shared.at[pl.ds(base + (ROWS_PER_TILE // GSZ) * GSZ, rem)])
    @pl.when(s == NS - 1)
    def _zero_tail():
        pltpu.sync_copy(a0.at[pl.ds(0, N - NS * ROWS_PER_TILE)],
                        h_shared.at[pl.ds(NS * ROWS_PER_TILE, N - NS * ROWS_PER_TILE)])

    if compute_deg:
        def zdeg(i, _):
            deg_v[pl.ds(pl.multiple_of(i * 16, 16), 16)] = zeros16
            return 0
        lax.fori_loop(0, N // 16, zdeg, 0)

    plsc.subcore_barrier()

    start = wid * GMAX
    nb = jnp.where(wid < NW - 1, GMAX // BODY, GTAIL // BODY)

    slots = ((a0, b0, dw0, sg0, ss0), (a1, b1, dw1, sg1, ss1))
    chunks = ((ic0, ii0), (ic1, ii1))

    def chunk_off(cc):
        return pl.multiple_of((start + cc * K) * GSZ, 8)

    def issue_ic(cc, ch):
        ic, sem = ch
        pltpu.async_copy(edge_hbm.at[:, pl.ds(chunk_off(cc), CH)], ic, sem)

    def wait_ic(cc, ch):
        ic, sem = ch
        pltpu.make_async_copy(edge_hbm.at[:, pl.ds(chunk_off(cc), CH)], ic, sem).wait()

    def issue_g(ch, jj, sl):
        ic, _ = ch
        a_buf, b_buf, _, sg, _ = sl
        pltpu.async_copy(a_hbm.at[ic.at[1, pl.ds(jj * GSZ, GSZ)]], a_buf, sg)
        pltpu.async_copy(b_hbm.at[ic.at[0, pl.ds(jj * GSZ, GSZ)]], b_buf, sg)

    def wait_g(ch, jj, sl):
        ic, _ = ch
        a_buf, b_buf, _, sg, _ = sl
        pltpu.make_async_copy(a_hbm.at[ic.at[1, pl.ds(jj * GSZ, GSZ)]], a_buf, sg).wait()
        pltpu.make_async_copy(b_hbm.at[ic.at[0, pl.ds(jj * GSZ, GSZ)]], b_buf, sg).wait()

    def wait_s(sl):
        a_buf, _, dw, _, ss = sl
        pltpu.make_async_copy(a_buf, h_shared.at[dw], ss).wait()

    def fill_dw(ch, jj, sl):
        ic, _ = ch
        dw = sl[2]
        for v in range(GSZ // 16):
            idx16 = ic[1, pl.ds(jj * GSZ + v * 16, 16)]
            dw[pl.ds(v * 16, 16)] = idx16
            if compute_deg:
                plsc.addupdate_scatter(deg_v, [idx16], ones16)

    def compute_scatter(sl):
        a_buf, b_buf, dw, _, ss = sl
        def row4(r, _):
            for rr in range(4):
                for v in range(VECS):
                    s_ = pl.ds(v * 16, 16)
                    a_buf[4 * r + rr, s_] = jnp.maximum(
                        a_buf[4 * r + rr, s_] + b_buf[4 * r + rr, s_], 0.0)
            return 0
        lax.fori_loop(0, GSZ // 4, row4, 0)
        pltpu.async_copy(a_buf, h_shared.at[dw], ss, add=True)

    issue_ic(0, chunks[0])
    issue_ic(1, chunks[1])
    wait_ic(0, chunks[0])
    issue_g(chunks[0], 0, slots[0])

    def body(q, _):
        for j in range(BODY):
            ch = chunks[0] if j < K else chunks[1]
            jj = j % K
            sl = slots[j % 2]
            wait_g(ch, jj, sl)
            if j == K:
                @pl.when(q + 1 < nb)
                def _pfc0():
                    issue_ic(2 * q + 2, chunks[0])
            fill_dw(ch, jj, sl)
            if j + 1 < BODY:
                nsl = slots[(j + 1) % 2]
                if j == 0:
                    @pl.when(q > 0)
                    def _dr():
                        wait_s(nsl)
                else:
                    wait_s(nsl)
                nch = chunks[0] if j + 1 < K else chunks[1]
                issue_g(nch, (j + 1) % K, nsl)
            else:
                @pl.when(q + 1 < nb)
                def _nextbody():
                    wait_ic(2 * q + 2, chunks[0])
                    wait_s(slots[0])
                    issue_g(chunks[0], 0, slots[0])
            compute_scatter(sl)
        @pl.when(q + 1 < nb)
        def _pfc1():
            issue_ic(2 * q + 3, chunks[1])
        return 0

    lax.fori_loop(0, nb, body, 0)

    @pl.when(wid == NW - 1)
    def _epilogue():
        issue_ic(4, chunks[0])
        wait_ic(4, chunks[0])
        for j in range(K):
            sl = slots[j % 2]
            wait_s(sl)
            issue_g(chunks[0], j, sl)
            wait_g(chunks[0], j, sl)
            fill_dw(chunks[0], j, sl)
            compute_scatter(sl)

    wait_s(slots[0])
    wait_s(slots[1])

    plsc.subcore_barrier()
    pltpu.sync_copy(h_shared.at[pl.ds(base, ROWS_PER_TILE)],
                    h_out.at[c].at[pl.ds(base, ROWS_PER_TILE)])
    @pl.when(s == NS - 1)
    def _write_tail():
        tail = N - NS * ROWS_PER_TILE
        pltpu.sync_copy(h_shared.at[pl.ds(NS * ROWS_PER_TILE, tail)],
                        h_out.at[c].at[pl.ds(NS * ROWS_PER_TILE, tail)])
    if compute_deg:
        pltpu.sync_copy(deg_v, deg_out.at[wid])


def _make_edge_kernel(compute_deg):
    mesh = plsc.VectorSubcoreMesh(core_axis_name="c", subcore_axis_name="s",
                                  num_cores=NC, num_subcores=NS)
    out_type = [jax.ShapeDtypeStruct((NC, N, D), jnp.float32)]
    if compute_deg:
        out_type.append(jax.ShapeDtypeStruct((NW, N), jnp.float32))
    scratch = [
        pltpu.VMEM((2, CH), jnp.int32),
        pltpu.VMEM((2, CH), jnp.int32),
        pltpu.VMEM((GSZ, D), jnp.float32),
        pltpu.VMEM((GSZ, D), jnp.float32),
        pltpu.VMEM((GSZ, D), jnp.float32),
        pltpu.VMEM((GSZ, D), jnp.float32),
        pltpu.VMEM((GSZ,), jnp.int32),
        pltpu.VMEM((GSZ,), jnp.int32),
    ]
    if compute_deg:
        scratch.append(pltpu.VMEM((N,), jnp.float32))
    scratch.append(pltpu.VMEM_SHARED((N, D), jnp.float32))
    scratch += [pltpu.SemaphoreType.DMA] * 6
    return pl.kernel(
        functools.partial(_edge_body, compute_deg),
        out_type=tuple(out_type) if compute_deg else out_type[0],
        mesh=mesh,
        scratch_types=scratch,
        compiler_params=pltpu.CompilerParams(needs_layout_passes=False),
    )


_BM = 1000


def _proj_body(x_ref, w_ref, b_ref, oa_ref, ob_ref):
    p = jnp.dot(x_ref[...], w_ref[...],
                preferred_element_type=jnp.float32,
                precision=lax.Precision.HIGHEST) + b_ref[...]
    oa_ref[...] = p[:, :D]
    ob_ref[...] = p[:, D:]


def _proj(x, w, bias):
    return pl.pallas_call(
        _proj_body,
        grid=(N // _BM,),
        in_specs=[
            pl.BlockSpec((_BM, D), lambda i: (i, 0)),
            pl.BlockSpec((D, 2 * D), lambda i: (0, 0)),
            pl.BlockSpec((1, 2 * D), lambda i: (0, 0)),
        ],
        out_specs=[pl.BlockSpec((_BM, D), lambda i: (i, 0))] * 2,
        out_shape=[jax.ShapeDtypeStruct((N, D), jnp.float32)] * 2,
    )(x, w, bias.reshape(1, 2 * D))


def _combine_body(split, h_ref, deg_ref, w_ref, u_ref, b_ref, *o_refs):
    hs = h_ref[0] + h_ref[1]
    degsum = jnp.sum(deg_ref[...], axis=0)
    p = (jnp.dot(hs, w_ref[...],
                 preferred_element_type=jnp.float32,
                 precision=lax.Precision.HIGHEST)
         + degsum[:, None] * u_ref[...]
         + b_ref[...])
    if split:
        o_refs[0][...] = p[:, :D]
        o_refs[1][...] = p[:, D:]
    else:
        o_refs[0][...] = p


def _combine(h_stack, deg_parts, w, u, bias):
    k = w.shape[1]
    split = k == 2 * D
    out_specs = [pl.BlockSpec((N, D), lambda i: (0, 0))]
    out_shape = [jax.ShapeDtypeStruct((N, D), jnp.float32)]
    if split:
        out_specs = out_specs * 2
        out_shape = out_shape * 2
    res = pl.pallas_call(
        functools.partial(_combine_body, split),
        grid=(1,),
        in_specs=[
            pl.BlockSpec((NC, N, D), lambda i: (0, 0, 0)),
            pl.BlockSpec((NW, N), lambda i: (0, 0)),
            pl.BlockSpec((D, k), lambda i: (0, 0)),
            pl.BlockSpec((1, k), lambda i: (0, 0)),
            pl.BlockSpec((1, k), lambda i: (0, 0)),
        ],
        out_specs=out_specs,
        out_shape=out_shape,
    )(h_stack, deg_parts, w, u.reshape(1, k), bias.reshape(1, k))
    return res if split else res[0]


def kernel(x, edge_index, W1_0, b1_0, W2_0, b2_0, W1_1, b1_1, W2_1, b2_1):
    Wa0, Wb0 = W1_0[:D], W1_0[D:]
    Wc0 = jnp.concatenate([Wa0 - Wb0, Wb0], axis=1)
    bias0 = jnp.concatenate([b1_0, jnp.zeros_like(b1_0)])
    A0, B0 = _proj(x, Wc0, bias0)

    H0, deg_parts = _make_edge_kernel(True)(A0, B0, edge_index)

    Wa1, Wb1 = W1_1[:D], W1_1[D:]
    Wcat1 = jnp.concatenate([Wa1 - Wb1, Wb1], axis=1)
    Wc1 = W2_0 @ Wcat1
    u1 = b2_0 @ Wcat1
    bias1 = jnp.concatenate([b1_1, jnp.zeros_like(b1_1)])
    A1, B1 = _combine(H0, deg_parts, Wc1, u1, bias1)

    H1 = _make_edge_kernel(False)(A1, B1, edge_index)

    return _combine(H1, deg_parts, W2_1, b2_1, jnp.zeros_like(b2_1))


# speedup vs baseline: 1.0212x; 1.0041x over previous
import functools

import jax
import jax.numpy as jnp
from jax import lax
from jax.experimental import pallas as pl
from jax.experimental.pallas import tpu as pltpu
from jax.experimental.pallas import tpu_sc as plsc

N = 10000
E = 320000
D = 128
GSZ = 64
G = E // GSZ
NC = 2
NS = 16
NW = NC * NS
ROWS_PER_TILE = (N // NS) // 8 * 8
VECS = D // 16


K = 8
CH = K * GSZ
BODY = 2 * K
GMAX = 160
GTAIL = G - (NW - 1) * GMAX


def _edge_body(compute_deg, a_hbm, b_hbm, edge_hbm, *refs):
    if compute_deg:
        (h_out, deg_out, ic0, ic1, a0, b0, a1, b1, dw0, dw1,
         deg_v, h_shared, ii0, ii1, sg0, sg1, ss0, ss1) = refs
    else:
        (h_out, ic0, ic1, a0, b0, a1, b1, dw0, dw1,
         h_shared, ii0, ii1, sg0, sg1, ss0, ss1) = refs
        deg_out = deg_v = None

    c = lax.axis_index("c")
    s = lax.axis_index("s")
    wid = s * NC + c

    zeros16 = jnp.zeros((16,), jnp.float32)
    ones16 = jnp.ones((16,), jnp.float32)

    def zrow(r, _):
        for v in range(VECS):
            a0[r, pl.ds(v * 16, 16)] = zeros16
        return 0
    lax.fori_loop(0, GSZ, zrow, 0)
    base = pl.multiple_of(s * ROWS_PER_TILE, 8)
    for k in range(ROWS_PER_TILE // GSZ):
        pltpu.sync_copy(a0, h_shared.at[pl.ds(base + k * GSZ, GSZ)])
    rem = ROWS_PER_TILE % GSZ
    if rem:
        pltpu.sync_copy(a0.at[pl.ds(0, rem)],
                        h_shared.at[pl.ds(base + (ROWS_PER_TILE // GSZ) * GSZ, rem)])
    @pl.when(s == NS - 1)
    def _zero_tail():
        pltpu.sync_copy(a0.at[pl.ds(0, N - NS * ROWS_PER_TILE)],
                        h_shared.at[pl.ds(NS * ROWS_PER_TILE, N - NS * ROWS_PER_TILE)])

    if compute_deg:
        def zdeg(i, _):
            deg_v[pl.ds(pl.multiple_of(i * 16, 16), 16)] = zeros16
            return 0
        lax.fori_loop(0, N // 16, zdeg, 0)

    plsc.subcore_barrier()

    start = wid * GMAX
    nb = jnp.where(wid < NW - 1, GMAX // BODY, GTAIL // BODY)

    slots = ((a0, b0, dw0, sg0, ss0), (a1, b1, dw1, sg1, ss1))
    chunks = ((ic0, ii0), (ic1, ii1))

    def chunk_off(cc):
        return pl.multiple_of((start + cc * K) * GSZ, 8)

    def issue_ic(cc, ch):
        ic, sem = ch
        pltpu.async_copy(edge_hbm.at[:, pl.ds(chunk_off(cc), CH)], ic, sem)

    def wait_ic(cc, ch):
        ic, sem = ch
        pltpu.make_async_copy(edge_hbm.at[:, pl.ds(chunk_off(cc), CH)], ic, sem).wait()

    def issue_g(ch, jj, sl):
        ic, _ = ch
        a_buf, b_buf, _, sg, _ = sl
        pltpu.async_copy(a_hbm.at[ic.at[1, pl.ds(jj * GSZ, GSZ)]], a_buf, sg)
        pltpu.async_copy(b_hbm.at[ic.at[0, pl.ds(jj * GSZ, GSZ)]], b_buf, sg)

    def wait_g(ch, jj, sl):
        ic, _ = ch
        a_buf, b_buf, _, sg, _ = sl
        pltpu.make_async_copy(a_hbm.at[ic.at[1, pl.ds(jj * GSZ, GSZ)]], a_buf, sg).wait()
        pltpu.make_async_copy(b_hbm.at[ic.at[0, pl.ds(jj * GSZ, GSZ)]], b_buf, sg).wait()

    def wait_s(sl):
        a_buf, _, dw, _, ss = sl
        pltpu.make_async_copy(
            a_buf, h_shared.at[ic0.at[1, pl.ds(0, GSZ)]], ss).wait()

    def fill_dw(ch, jj, sl):
        ic, _ = ch
        if compute_deg:
            for v in range(GSZ // 16):
                idx16 = ic[1, pl.ds(jj * GSZ + v * 16, 16)]
                plsc.addupdate_scatter(deg_v, [idx16], ones16)

    def compute_scatter(ch, jj, sl):
        ic, _ = ch
        a_buf, b_buf, dw, _, ss = sl
        def row4(r, _):
            for rr in range(4):
                for v in range(VECS):
                    s_ = pl.ds(v * 16, 16)
                    a_buf[4 * r + rr, s_] = jnp.maximum(
                        a_buf[4 * r + rr, s_] + b_buf[4 * r + rr, s_], 0.0)
            return 0
        lax.fori_loop(0, GSZ // 4, row4, 0)
        pltpu.async_copy(a_buf, h_shared.at[ic.at[1, pl.ds(jj * GSZ, GSZ)]], ss, add=True)

    issue_ic(0, chunks[0])
    issue_ic(1, chunks[1])
    wait_ic(0, chunks[0])
    issue_g(chunks[0], 0, slots[0])

    def body(q, _):
        for j in range(BODY):
            ch = chunks[0] if j < K else chunks[1]
            jj = j % K
            sl = slots[j % 2]
            wait_g(ch, jj, sl)
            if j == K + 1:
                @pl.when(q + 1 < nb)
                def _pfc0():
                    issue_ic(2 * q + 2, chunks[0])
            fill_dw(ch, jj, sl)
            if j + 1 < BODY:
                nsl = slots[(j + 1) % 2]
                if j == 0:
                    @pl.when(q > 0)
                    def _dr():
                        wait_s(nsl)
                        issue_ic(2 * q + 1, chunks[1])
                else:
                    wait_s(nsl)
                if j + 1 == K:
                    wait_ic(2 * q + 1, chunks[1])
                nch = chunks[0] if j + 1 < K else chunks[1]
                issue_g(nch, (j + 1) % K, nsl)
            else:
                @pl.when(q + 1 < nb)
                def _nextbody():
                    wait_ic(2 * q + 2, chunks[0])
                    wait_s(slots[0])
                    issue_g(chunks[0], 0, slots[0])
            compute_scatter(ch, jj, sl)
        return 0

    lax.fori_loop(0, nb, body, 0)

    @pl.when(wid == NW - 1)
    def _epilogue():
        issue_ic(4, chunks[0])
        wait_ic(4, chunks[0])
        for j in range(K):
            sl = slots[j % 2]
            wait_s(sl)
            issue_g(chunks[0], j, sl)
            wait_g(chunks[0], j, sl)
            fill_dw(chunks[0], j, sl)
            compute_scatter(chunks[0], j, sl)

    wait_s(slots[0])
    wait_s(slots[1])

    plsc.subcore_barrier()
    pltpu.sync_copy(h_shared.at[pl.ds(base, ROWS_PER_TILE)],
                    h_out.at[c].at[pl.ds(base, ROWS_PER_TILE)])
    @pl.when(s == NS - 1)
    def _write_tail():
        tail = N - NS * ROWS_PER_TILE
        pltpu.sync_copy(h_shared.at[pl.ds(NS * ROWS_PER_TILE, tail)],
                        h_out.at[c].at[pl.ds(NS * ROWS_PER_TILE, tail)])
    if compute_deg:
        pltpu.sync_copy(deg_v, deg_out.at[wid])


def _make_edge_kernel(compute_deg):
    mesh = plsc.VectorSubcoreMesh(core_axis_name="c", subcore_axis_name="s",
                                  num_cores=NC, num_subcores=NS)
    out_type = [jax.ShapeDtypeStruct((NC, N, D), jnp.float32)]
    if compute_deg:
        out_type.append(jax.ShapeDtypeStruct((NW, N), jnp.float32))
    scratch = [
        pltpu.VMEM((2, CH), jnp.int32),
        pltpu.VMEM((2, CH), jnp.int32),
        pltpu.VMEM((GSZ, D), jnp.float32),
        pltpu.VMEM((GSZ, D), jnp.float32),
        pltpu.VMEM((GSZ, D), jnp.float32),
        pltpu.VMEM((GSZ, D), jnp.float32),
        pltpu.VMEM((GSZ,), jnp.int32),
        pltpu.VMEM((GSZ,), jnp.int32),
    ]
    if compute_deg:
        scratch.append(pltpu.VMEM((N,), jnp.float32))
    scratch.append(pltpu.VMEM_SHARED((N, D), jnp.float32))
    scratch += [pltpu.SemaphoreType.DMA] * 6
    return pl.kernel(
        functools.partial(_edge_body, compute_deg),
        out_type=tuple(out_type) if compute_deg else out_type[0],
        mesh=mesh,
        scratch_types=scratch,
        compiler_params=pltpu.CompilerParams(needs_layout_passes=False),
    )


_BM = 1000


def _proj_body(x_ref, w_ref, b_ref, oa_ref, ob_ref):
    p = jnp.dot(x_ref[...], w_ref[...],
                preferred_element_type=jnp.float32,
                precision=lax.Precision.HIGHEST) + b_ref[...]
    oa_ref[...] = p[:, :D]
    ob_ref[...] = p[:, D:]


def _proj(x, w, bias):
    return pl.pallas_call(
        _proj_body,
        grid=(N // _BM,),
        in_specs=[
            pl.BlockSpec((_BM, D), lambda i: (i, 0)),
            pl.BlockSpec((D, 2 * D), lambda i: (0, 0)),
            pl.BlockSpec((1, 2 * D), lambda i: (0, 0)),
        ],
        out_specs=[pl.BlockSpec((_BM, D), lambda i: (i, 0))] * 2,
        out_shape=[jax.ShapeDtypeStruct((N, D), jnp.float32)] * 2,
    )(x, w, bias.reshape(1, 2 * D))


def _combine_body(split, h_ref, deg_ref, w_ref, u_ref, b_ref, *o_refs):
    hs = h_ref[0] + h_ref[1]
    degsum = jnp.sum(deg_ref[...], axis=0)
    p = (jnp.dot(hs, w_ref[...],
                 preferred_element_type=jnp.float32,
                 precision=lax.Precision.HIGHEST)
         + degsum[:, None] * u_ref[...]
         + b_ref[...])
    if split:
        o_refs[0][...] = p[:, :D]
        o_refs[1][...] = p[:, D:]
    else:
        o_refs[0][...] = p


def _combine(h_stack, deg_parts, w, u, bias):
    k = w.shape[1]
    split = k == 2 * D
    out_specs = [pl.BlockSpec((N, D), lambda i: (0, 0))]
    out_shape = [jax.ShapeDtypeStruct((N, D), jnp.float32)]
    if split:
        out_specs = out_specs * 2
        out_shape = out_shape * 2
    res = pl.pallas_call(
        functools.partial(_combine_body, split),
        grid=(1,),
        in_specs=[
            pl.BlockSpec((NC, N, D), lambda i: (0, 0, 0)),
            pl.BlockSpec((NW, N), lambda i: (0, 0)),
            pl.BlockSpec((D, k), lambda i: (0, 0)),
            pl.BlockSpec((1, k), lambda i: (0, 0)),
            pl.BlockSpec((1, k), lambda i: (0, 0)),
        ],
        out_specs=out_specs,
        out_shape=out_shape,
    )(h_stack, deg_parts, w, u.reshape(1, k), bias.reshape(1, k))
    return res if split else res[0]


def kernel(x, edge_index, W1_0, b1_0, W2_0, b2_0, W1_1, b1_1, W2_1, b2_1):
    Wa0, Wb0 = W1_0[:D], W1_0[D:]
    Wc0 = jnp.concatenate([Wa0 - Wb0, Wb0], axis=1)
    bias0 = jnp.concatenate([b1_0, jnp.zeros_like(b1_0)])
    A0, B0 = _proj(x, Wc0, bias0)

    H0, deg_parts = _make_edge_kernel(True)(A0, B0, edge_index)

    Wa1, Wb1 = W1_1[:D], W1_1[D:]
    Wcat1 = jnp.concatenate([Wa1 - Wb1, Wb1], axis=1)
    Wc1 = W2_0 @ Wcat1
    u1 = b2_0 @ Wcat1
    bias1 = jnp.concatenate([b1_1, jnp.zeros_like(b1_1)])
    A1, B1 = _combine(H0, deg_parts, Wc1, u1, bias1)

    H1 = _make_edge_kernel(False)(A1, B1, edge_index)

    return _combine(H1, deg_parts, W2_1, b2_1, jnp.zeros_like(b2_1))
